# baseline (speedup 1.0000x reference)
import functools

import jax
import jax.numpy as jnp
from jax import lax
from jax.experimental import pallas as pl
from jax.experimental.pallas import tpu as pltpu
from jax.experimental.pallas import tpu_sc as plsc

B = 16384
D = 64
L = 16
NC = 2
NS = 16
NW = NC * NS
BPW = B // NW
CH = 64
NCHUNK = BPW // CH

_BITREV4 = [0, 8, 4, 12, 2, 10, 6, 14, 1, 9, 5, 13, 3, 11, 7, 15]

_DNUMS = lax.GatherDimensionNumbers(
    offset_dims=(), collapsed_slice_dims=(0,), start_index_map=(0,))


def _shuffle(p, perm):
    return lax.gather(p, perm, _DNUMS, (1,),
                      mode=lax.GatherScatterMode.PROMISE_IN_BOUNDS)


def _gmf_body(user_hbm, item_hbm, uemb2_hbm, iemb2_hbm, w_hbm, bias_hbm,
              out_hbm,
              uidx_v, iidx_v, uhalf_v, ihalf_v, upar_v, ipar_v,
              ubuf0, ubuf1, ibuf0, ibuf1, w_v, bias_v, out_v,
              su0, su1, si0, si1):
    wid = lax.axis_index("s") * NC + lax.axis_index("c")
    base = wid * BPW

    pltpu.sync_copy(user_hbm.at[pl.ds(base, BPW)], uidx_v)
    pltpu.sync_copy(item_hbm.at[pl.ds(base, BPW)], iidx_v)
    pltpu.sync_copy(w_hbm, w_v)
    pltpu.sync_copy(bias_hbm, bias_v)

    def split_idx(i, _):
        u = uidx_v[pl.ds(i * L, L)]
        it = iidx_v[pl.ds(i * L, L)]
        uhalf_v[pl.ds(i * L, L)] = u >> 1
        ihalf_v[pl.ds(i * L, L)] = it >> 1
        upar_v[pl.ds(i * L, L)] = (u & 1) << 6
        ipar_v[pl.ds(i * L, L)] = (it & 1) << 6
        return _

    lax.fori_loop(0, BPW // L, split_idx, 0)

    ubufs = (ubuf0, ubuf1)
    ibufs = (ibuf0, ibuf1)
    usems = (su0, su1)
    isems = (si0, si1)

    def fire(c):
        p = c % 2
        cu = pltpu.async_copy(uemb2_hbm.at[uhalf_v.at[pl.ds(c * CH, CH)]],
                              ubufs[p], usems[p])
        ci = pltpu.async_copy(iemb2_hbm.at[ihalf_v.at[pl.ds(c * CH, CH)]],
                              ibufs[p], isems[p])
        return cu, ci

    wv = tuple(w_v[pl.ds(j * L, L)] for j in range(4))
    bias = bias_v[...]
    lane = lax.broadcasted_iota(jnp.int32, (L,), 0)
    perms = tuple((lane ^ (8 >> s))[:, None] for s in range(4))
    masks = tuple((lane & (8 >> s)) == 0 for s in range(4))

    def merge(a, b, lvl):
        m = masks[lvl]
        return (jnp.where(m, a, _shuffle(b, perms[lvl]))
                + jnp.where(m, _shuffle(a, perms[lvl]), b))

    def make_group(ub, ib, c):
        def group(g, _):
            upar = upar_v[pl.ds(c * CH + g * L, L)]
            ipar = ipar_v[pl.ds(c * CH + g * L, L)]
            stack = []
            for t in range(L):
                r = _BITREV4[t]
                k = g * L + r
                rsplat = jnp.full((L,), r, jnp.int32)
                uoff = _shuffle(upar, rsplat[:, None]) + lane
                ioff = _shuffle(ipar, rsplat[:, None]) + lane
                ksplat = rsplat + (g * L)
                v = None
                for j in range(4):
                    term = (plsc.load_gather(ub, [ksplat, uoff + (j * L)])
                            * plsc.load_gather(ib, [ksplat, ioff + (j * L)])
                            * wv[j])
                    v = term if v is None else v + term
                lvl = 0
                while stack and stack[-1][0] == lvl:
                    pv = stack.pop()[1]
                    v = merge(pv, v, lvl)
                    lvl += 1
                stack.append((lvl, v))
            out_v[pl.ds(c * CH + g * L, L)] = stack[0][1] + bias
            return _
        return group

    handles = fire(0)
    for c in range(NCHUNK):
        nxt = fire(c + 1) if c + 1 < NCHUNK else None
        handles[0].wait()
        handles[1].wait()
        p = c % 2
        lax.fori_loop(0, CH // L, make_group(ubufs[p], ibufs[p], c), 0)
        handles = nxt

    pltpu.sync_copy(out_v, out_hbm.at[pl.ds(base, BPW)])


@jax.jit
def kernel(user, item, mf_user_embed, mf_item_embed, final_w, final_b):
    NB_USERS, NB_ITEMS = mf_user_embed.shape[0], mf_item_embed.shape[0]
    uemb2 = mf_user_embed.reshape(NB_USERS // 2, 2 * D)
    iemb2 = mf_item_embed.reshape(NB_ITEMS // 2, 2 * D)
    w_flat = final_w.reshape(D)
    bias16 = jnp.tile(final_b.reshape(1), L)
    mesh = plsc.VectorSubcoreMesh(core_axis_name="c", subcore_axis_name="s")
    run = functools.partial(
        pl.kernel,
        mesh=mesh,
        compiler_params=pltpu.CompilerParams(use_tc_tiling_on_sc=True,
                                             needs_layout_passes=False),
        out_type=jax.ShapeDtypeStruct((B,), jnp.float32),
        scratch_types=[
            pltpu.VMEM((BPW,), jnp.int32),
            pltpu.VMEM((BPW,), jnp.int32),
            pltpu.VMEM((BPW,), jnp.int32),
            pltpu.VMEM((BPW,), jnp.int32),
            pltpu.VMEM((BPW,), jnp.int32),
            pltpu.VMEM((BPW,), jnp.int32),
            pltpu.VMEM((CH, 2 * D), jnp.float32),
            pltpu.VMEM((CH, 2 * D), jnp.float32),
            pltpu.VMEM((CH, 2 * D), jnp.float32),
            pltpu.VMEM((CH, 2 * D), jnp.float32),
            pltpu.VMEM((D,), jnp.float32),
            pltpu.VMEM((L,), jnp.float32),
            pltpu.VMEM((BPW,), jnp.float32),
            pltpu.SemaphoreType.DMA,
            pltpu.SemaphoreType.DMA,
            pltpu.SemaphoreType.DMA,
            pltpu.SemaphoreType.DMA,
        ],
    )(_gmf_body)
    out = run(user.astype(jnp.int32), item.astype(jnp.int32),
              uemb2, iemb2, w_flat, bias16)
    return out.reshape(B, 1)


# speedup vs baseline: 2.6323x; 2.6323x over previous
import functools

import jax
import jax.numpy as jnp
from jax import lax
from jax.experimental import pallas as pl
from jax.experimental.pallas import tpu as pltpu
from jax.experimental.pallas import tpu_sc as plsc

B = 16384
D = 64
L = 16
NC = 2
NS = 16
NW = NC * NS

NB_USERS = 1000000
NB_ITEMS = 100000
UQ = (NB_USERS + 127) // 128
IQ = (NB_ITEMS + 127) // 128
UPT = (UQ + NW - 1) // NW
IPT = (IQ + NW - 1) // NW
WC = 3
UWIN = (UPT + WC - 1) // WC
IWIN = (IPT + WC - 1) // WC
NSCAN = B // L
LCAP = B + L
PREF = 128

_DNUMS = lax.GatherDimensionNumbers(
    offset_dims=(), collapsed_slice_dims=(0,), start_index_map=(0,))


def _shuffle(p, perm):
    return lax.gather(p, perm, _DNUMS, (1,),
                      mode=lax.GatherScatterMode.PROMISE_IN_BOUNDS)


def _lane():
    return lax.broadcasted_iota(jnp.int32, (L,), 0)


def _lanesum(p):
    lane = _lane()
    for s in range(4):
        p = p + _shuffle(p, (lane ^ (8 >> s))[:, None])
    return p


def _splat_at(ref, e):
    return plsc.load_gather(ref, [jnp.zeros((L,), jnp.int32) + e])


def _scan_matches(idx_v, vallist, blist, lo, hi):
    lane = _lane()

    def step(i, cnt):
        v = idx_v[pl.ds(i * L, L)]
        q = v >> 7
        m = (q >= lo) & (q < hi)
        plsc.store_compressed(vallist.at[pl.ds(cnt, L)], v, mask=m)
        plsc.store_compressed(blist.at[pl.ds(cnt, L)], (i * L) + lane, mask=m)
        return cnt + jnp.max(plsc.all_reduce_population_count(m))

    return lax.fori_loop(0, NSCAN, step, jnp.int32(0))


def _window_scan(vallist, blist, winlist, cnt, wlo):
    lane = _lane()
    nv = (cnt + L - 1) // L

    def step(g, wcnt):
        v = vallist[pl.ds(g * L, L)]
        b = blist[pl.ds(g * L, L)]
        q = v >> 7
        m = (q >= wlo) & (q < wlo + WC) & ((g * L + lane) < cnt)
        packed = ((q - wlo) << 21) | ((v & 127) << 14) | b
        plsc.store_compressed(winlist.at[pl.ds(wcnt, L)], packed, mask=m)
        return wcnt + jnp.max(plsc.all_reduce_population_count(m))

    return lax.fori_loop(0, nv, step, jnp.int32(0))


def _minor_of(ps):
    return ((ps >> 21) << 7) | ((ps >> 14) & 127)


_CVEC = None


def _item_body(item_hbm, iembT_hbm, w_hbm, itemw_hbm,
               iidx_v, slist, blist, winlist, winbuf, stag, w_v,
               sem_w, sem_s):
    wid = lax.axis_index("s") * NC + lax.axis_index("c")
    lo = wid * IPT
    hi = jnp.minimum(lo + IPT, IQ)

    pltpu.sync_copy(item_hbm, iidx_v)
    pltpu.sync_copy(w_hbm, w_v)
    wv = tuple(w_v[pl.ds(j * L, L)] for j in range(4))
    lane = _lane()
    cvec = tuple(lane + j * L for j in range(4))

    cnt = _scan_matches(iidx_v, slist, blist, lo, hi)

    def window(w, _):
        wlo = lo + w * WC
        for ci in range(WC):
            q = jnp.minimum(wlo + ci, IQ - 1)
            pltpu.async_copy(
                iembT_hbm.at[:, pl.ds(pl.multiple_of(q * 128, 128), 128)],
                winbuf.at[:, pl.ds(ci * 128, 128)], sem_w)
        wcnt = _window_scan(slist, blist, winlist, cnt, wlo)
        npf = jnp.minimum(wcnt, PREF)
        pltpu.make_async_copy(iembT_hbm.at[:, pl.ds(0, WC * 128)], winbuf,
                              sem_w).wait()

        def fast(e, _):
            ps = _splat_at(winlist, e)
            minor = _minor_of(ps)
            for j in range(4):
                g = plsc.load_gather(winbuf, [cvec[j], minor]) * wv[j]
                stag[pl.ds(e * D + j * L, L)] = g
            b = jnp.max(ps) & 16383
            pltpu.async_copy(stag.at[pl.ds(e * D, D)],
                             itemw_hbm.at[pl.ds(b * D, D)], sem_s)
            return _

        lax.fori_loop(0, npf, fast, 0)

        def slow(e, _):
            ps = _splat_at(winlist, e)
            minor = _minor_of(ps)
            for j in range(4):
                g = plsc.load_gather(winbuf, [cvec[j], minor]) * wv[j]
                stag[pl.ds(j * L, L)] = g
            b = jnp.max(ps) & 16383
            pltpu.async_copy(stag.at[pl.ds(0, D)],
                             itemw_hbm.at[pl.ds(b * D, D)], sem_s).wait()
            return _

        lax.fori_loop(npf, wcnt, slow, 0)

        def drain(e, _):
            pltpu.make_async_copy(itemw_hbm.at[pl.ds(0, D)],
                                  stag.at[pl.ds(0, D)], sem_s).wait()
            return _

        lax.fori_loop(0, npf, drain, 0)
        return _

    lax.fori_loop(0, IWIN, window, 0)


def _user_body(user_hbm, uembT_hbm, itemw_hbm, bias_hbm, outw_hbm,
               uidx_v, rlist, blist, winlist, winbuf0, winbuf1, irows, vals,
               bias_v, sw0, sw1, sem_r, sem_o):
    wid = lax.axis_index("s") * NC + lax.axis_index("c")
    lo = wid * UPT
    hi = jnp.minimum(lo + UPT, UQ)

    pltpu.sync_copy(user_hbm, uidx_v)
    pltpu.sync_copy(bias_hbm, bias_v)
    bias = bias_v[...]
    lane = _lane()
    cvec = tuple(lane + j * L for j in range(4))

    cnt = _scan_matches(uidx_v, rlist, blist, lo, hi)

    bufs = (winbuf0, winbuf1)
    sems = (sw0, sw1)

    def fire_win(w, buf, sem):
        for ci in range(WC):
            q = jnp.minimum(lo + w * WC + ci, UQ - 1)
            pltpu.async_copy(
                uembT_hbm.at[:, pl.ds(pl.multiple_of(q * 128, 128), 128)],
                buf.at[:, pl.ds(ci * 128, 128)], sem)

    def process(w, buf, sem):
        wlo = lo + w * WC
        wcnt = _window_scan(rlist, blist, winlist, cnt, wlo)
        npf = jnp.minimum(wcnt, PREF)

        def pref(e, _):
            ps = _splat_at(winlist, e)
            b = jnp.max(ps) & 16383
            pltpu.async_copy(itemw_hbm.at[pl.ds(b * D, D)],
                             irows.at[pl.ds(e * D, D)], sem_r)
            return _

        lax.fori_loop(0, npf, pref, 0)
        pltpu.make_async_copy(uembT_hbm.at[:, pl.ds(0, WC * 128)], buf,
                              sem_w_dummy := sem).wait()

        def drain_r(e, _):
            pltpu.make_async_copy(itemw_hbm.at[pl.ds(0, D)],
                                  irows.at[pl.ds(0, D)], sem_r).wait()
            return _

        lax.fori_loop(0, npf, drain_r, 0)

        def fast(e, _):
            ps = _splat_at(winlist, e)
            minor = _minor_of(ps)
            acc = None
            for j in range(4):
                t = (plsc.load_gather(buf, [cvec[j], minor])
                     * irows[pl.ds(e * D + j * L, L)])
                acc = t if acc is None else acc + t
            val = _lanesum(acc) + bias
            vals[pl.ds(e * L, L)] = val
            b = jnp.max(ps) & 16383
            pltpu.async_copy(vals.at[pl.ds(e * L, L)],
                             outw_hbm.at[pl.ds(b * L, L)], sem_o)
            return _

        lax.fori_loop(0, npf, fast, 0)

        def slow(e, _):
            ps = _splat_at(winlist, e)
            minor = _minor_of(ps)
            b = jnp.max(ps) & 16383
            pltpu.async_copy(itemw_hbm.at[pl.ds(b * D, D)],
                             irows.at[pl.ds(0, D)], sem_r).wait()
            acc = None
            for j in range(4):
                t = (plsc.load_gather(buf, [cvec[j], minor])
                     * irows[pl.ds(j * L, L)])
                acc = t if acc is None else acc + t
            val = _lanesum(acc) + bias
            vals[pl.ds(0, L)] = val
            pltpu.async_copy(vals.at[pl.ds(0, L)],
                             outw_hbm.at[pl.ds(b * L, L)], sem_o).wait()
            return _

        lax.fori_loop(npf, wcnt, slow, 0)

        def drain_o(e, _):
            pltpu.make_async_copy(outw_hbm.at[pl.ds(0, L)],
                                  vals.at[pl.ds(0, L)], sem_o).wait()
            return _

        lax.fori_loop(0, npf, drain_o, 0)

    fire_win(0, winbuf0, sw0)
    fire_win(1, winbuf1, sw1)

    def pair(i, _):
        process(2 * i, winbuf0, sw0)

        @pl.when(2 * i + 2 < UWIN)
        def _f0():
            fire_win(2 * i + 2, winbuf0, sw0)

        process(2 * i + 1, winbuf1, sw1)

        @pl.when(2 * i + 3 < UWIN)
        def _f1():
            fire_win(2 * i + 3, winbuf1, sw1)

        return _

    lax.fori_loop(0, (UWIN + 1) // 2, pair, 0)


@jax.jit
def kernel(user, item, mf_user_embed, mf_item_embed, final_w, final_b):
    uembT = mf_user_embed.T
    iembT = mf_item_embed.T
    w_flat = final_w.reshape(D)
    bias16 = jnp.tile(final_b.reshape(1), L)
    mesh = plsc.VectorSubcoreMesh(core_axis_name="c", subcore_axis_name="s")
    cp = pltpu.CompilerParams(use_tc_tiling_on_sc=True,
                              needs_layout_passes=False)

    item_run = functools.partial(
        pl.kernel, mesh=mesh, compiler_params=cp,
        out_type=jax.ShapeDtypeStruct((B * D,), jnp.float32),
        scratch_types=[
            pltpu.VMEM((B,), jnp.int32),
            pltpu.VMEM((LCAP,), jnp.int32),
            pltpu.VMEM((LCAP,), jnp.int32),
            pltpu.VMEM((LCAP,), jnp.int32),
            pltpu.VMEM((D, WC * 128), jnp.float32),
            pltpu.VMEM((PREF * D,), jnp.float32),
            pltpu.VMEM((D,), jnp.float32),
            pltpu.SemaphoreType.DMA,
            pltpu.SemaphoreType.DMA,
        ],
    )(_item_body)
    itemw = item_run(item.astype(jnp.int32), iembT, w_flat)

    user_run = functools.partial(
        pl.kernel, mesh=mesh, compiler_params=cp,
        out_type=jax.ShapeDtypeStruct((B * L,), jnp.float32),
        scratch_types=[
            pltpu.VMEM((B,), jnp.int32),
            pltpu.VMEM((LCAP,), jnp.int32),
            pltpu.VMEM((LCAP,), jnp.int32),
            pltpu.VMEM((LCAP,), jnp.int32),
            pltpu.VMEM((D, WC * 128), jnp.float32),
            pltpu.VMEM((D, WC * 128), jnp.float32),
            pltpu.VMEM((PREF * D,), jnp.float32),
            pltpu.VMEM((PREF * L,), jnp.float32),
            pltpu.VMEM((L,), jnp.float32),
            pltpu.SemaphoreType.DMA,
            pltpu.SemaphoreType.DMA,
            pltpu.SemaphoreType.DMA,
            pltpu.SemaphoreType.DMA,
        ],
    )(_user_body)
    outw = user_run(user.astype(jnp.int32), uembT, itemw, bias16)

    return outw.reshape(B, L)[:, 0:1]
